# baseline (speedup 1.0000x reference)
import jax
import jax.numpy as jnp
from jax.experimental import pallas as pl

T, H, L, V, B, MAXN = 8192, 512, 128, 1024, 256, 15
R = 256
NB = T // R


def _body(idx_ref, ctx_ref, hnei_ref, tv_ref, emb_ref,
          wz_ref, wr_ref, ur_ref, wh_ref, w1_ref, w2_ref, wo_ref,
          ui_ref, u1_ref, u2_ref, bias_ref,
          word_ref, stop_ref):
    f32 = jnp.float32
    idx = idx_ref[0, 0, :]
    ctx = ctx_ref[0, 0, :]

    iota_v = jax.lax.broadcasted_iota(jnp.int32, (R, V), 1)
    oh_x = (idx[:, None] == iota_v).astype(f32)
    x = jnp.dot(oh_x, emb_ref[...], preferred_element_type=f32)

    iota_b = jax.lax.broadcasted_iota(jnp.int32, (R, B), 1)
    oh_c = (ctx[:, None] == iota_b).astype(f32)
    tc = jnp.dot(oh_c, tv_ref[...], preferred_element_type=f32)

    hnei = hnei_ref[...]
    sum_h = jnp.sum(hnei, axis=1)

    wz_b = bias_ref[0, :H]
    wr_b = bias_ref[1, :H]
    wh_b = bias_ref[2, :H]
    w_b = bias_ref[3, :H]
    ui_b = bias_ref[4, :H]
    u_b = bias_ref[5, :H]
    uo_row = bias_ref[6, :H]
    uo_b = bias_ref[7, 0]

    z_pre = (jnp.dot(x, wz_ref[:H, :], preferred_element_type=f32)
             + jnp.dot(sum_h, wz_ref[H:, :], preferred_element_type=f32)
             + wz_b[None, :])
    z = jax.nn.sigmoid(z_pre)

    r1 = jnp.dot(x, wr_ref[...], preferred_element_type=f32) + wr_b[None, :]
    r2 = jnp.dot(hnei.reshape(R * MAXN, H), ur_ref[...],
                 preferred_element_type=f32).reshape(R, MAXN, H)
    r = jax.nn.sigmoid(r1[:, None, :] + r2)
    sum_gated = jnp.sum(r * hnei, axis=1)

    pre_h = jnp.tanh(jnp.dot(x, wh_ref[:H, :], preferred_element_type=f32)
                     + jnp.dot(sum_gated, wh_ref[H:, :], preferred_element_type=f32)
                     + wh_b[None, :])
    new_h = (1.0 - z) * sum_h + z * pre_h

    wh_act = jax.nn.relu(jnp.dot(new_h, w1_ref[...], preferred_element_type=f32)
                         + jnp.dot(tc, w2_ref[...], preferred_element_type=f32)
                         + w_b[None, :])
    word = jnp.dot(wh_act, wo_ref[...], preferred_element_type=f32)
    word_ref[...] = word + bias_ref[8:8 + (V // H), :].reshape(1, V)

    sh = jax.nn.relu(jnp.dot(x, ui_ref[:H, :], preferred_element_type=f32)
                     + jnp.dot(sum_h, ui_ref[H:, :], preferred_element_type=f32)
                     + ui_b[None, :])
    sh2 = jax.nn.relu(jnp.dot(sh, u1_ref[...], preferred_element_type=f32)
                      + jnp.dot(tc, u2_ref[...], preferred_element_type=f32)
                      + u_b[None, :])
    stop = jnp.sum(sh2 * uo_row[None, :], axis=1, keepdims=True) + uo_b
    stop_ref[...] = jnp.broadcast_to(stop, (R, 128))


@jax.jit
def _run(cur_x_idx, contexts, cur_h_nei, tree_vecs, emb, Wz_w, Wz_b, Wr_w,
         Wr_b, Ur_w, Wh_w, Wh_b, W_w, W_b, Wo_w, Wo_b, Ui_w, Ui_b, U_w, U_b,
         Uo_w, Uo_b):
    f32 = jnp.float32
    idx2 = cur_x_idx.astype(jnp.int32).reshape(NB, 1, R)
    ctx2 = contexts.astype(jnp.int32).reshape(NB, 1, R)

    wz = Wz_w.T
    wr = Wr_w.T
    ur = Ur_w.T
    wh = Wh_w.T
    w1 = W_w.T[:H, :]
    w2 = W_w.T[H:, :]
    wo = Wo_w.T
    ui = Ui_w.T
    u1 = U_w.T[:H, :]
    u2 = U_w.T[H:, :]

    bias = jnp.stack([
        Wz_b, Wr_b, Wh_b, W_b, Ui_b, U_b, Uo_w[0, :],
        jnp.full((H,), Uo_b[0], f32),
    ], axis=0)
    bias = jnp.concatenate([bias, Wo_b.reshape(V // H, H)], axis=0)

    full = lambda shape: pl.BlockSpec(shape, lambda i: (0,) * len(shape))
    grid_spec = pl.GridSpec(
        grid=(NB,),
        in_specs=[
            pl.BlockSpec((1, 1, R), lambda i: (i, 0, 0)),
            pl.BlockSpec((1, 1, R), lambda i: (i, 0, 0)),
            pl.BlockSpec((R, MAXN, H), lambda i: (i, 0, 0)),
            full((B, L)),
            full((V, H)),
            full((2 * H, H)),
            full((H, H)),
            full((H, H)),
            full((2 * H, H)),
            full((H, H)),
            full((L, H)),
            full((H, V)),
            full((2 * H, H)),
            full((H, H)),
            full((L, H)),
            full((8 + V // H, H)),
        ],
        out_specs=[
            pl.BlockSpec((R, V), lambda i: (i, 0)),
            pl.BlockSpec((R, 128), lambda i: (i, 0)),
        ],
    )
    word, stop = pl.pallas_call(
        _body,
        grid_spec=grid_spec,
        out_shape=[
            jax.ShapeDtypeStruct((T, V), f32),
            jax.ShapeDtypeStruct((T, 128), f32),
        ],
    )(idx2, ctx2, cur_h_nei, tree_vecs, emb, wz, wr, ur, wh, w1, w2, wo,
      ui, u1, u2, bias)
    return jnp.concatenate([word, stop[:, :1]], axis=1)


def kernel(cur_x_idx, contexts, cur_h_nei, tree_vecs, emb, Wz_w, Wz_b, Wr_w,
           Wr_b, Ur_w, Wh_w, Wh_b, W_w, W_b, Wo_w, Wo_b, Ui_w, Ui_b, U_w,
           U_b, Uo_w, Uo_b):
    return _run(cur_x_idx, contexts, cur_h_nei, tree_vecs, emb, Wz_w, Wz_b,
                Wr_w, Wr_b, Ur_w, Wh_w, Wh_b, W_w, W_b, Wo_w, Wo_b, Ui_w,
                Ui_b, U_w, U_b, Uo_w, Uo_b)
